# Initial kernel scaffold; baseline (speedup 1.0000x reference)
import jax
import jax.numpy as jnp
from jax.experimental import pallas as pl


def kernel(x, emb_proto, emb_flags, emb_dir, w_len, b_len, w_iat, b_iat, w_fusion, b_fusion, gamma, beta):
    raise NotImplementedError("write your pallas kernel here")


# speedup vs baseline: 8.6835x; 8.6835x over previous
import jax
import jax.numpy as jnp
from jax.experimental import pallas as pl
from jax.experimental.pallas import tpu as pltpu


BT = 2048


def _body(x_ref, emb_proto_ref, emb_flags_ref, emb_dir_ref, u_len_ref,
          u_iat_ref, w_fusion_ref, b_fusion_ref, gamma_ref, beta_ref,
          o_ref, tp_hi, tp_lo, tf_hi, tf_lo, aux):
    step = pl.program_id(0)

    @pl.when(step == 0)
    def _build_tables():
        wf = w_fusion_ref[...]
        tp = jnp.dot(emb_proto_ref[...], wf[0:32, :],
                     preferred_element_type=jnp.float32)
        hi = tp.astype(jnp.bfloat16)
        tp_hi[...] = hi
        tp_lo[...] = (tp - hi.astype(jnp.float32)).astype(jnp.bfloat16)
        tf = jnp.dot(emb_flags_ref[...], wf[32:64, :],
                     preferred_element_type=jnp.float32)
        hi_f = tf.astype(jnp.bfloat16)
        tf_hi[...] = hi_f
        tf_lo[...] = (tf - hi_f.astype(jnp.float32)).astype(jnp.bfloat16)
        td = jnp.dot(emb_dir_ref[...], wf[64:72, :],
                     preferred_element_type=jnp.float32)
        ul = jnp.dot(u_len_ref[...], wf[72:104, :],
                     preferred_element_type=jnp.float32)
        ui = jnp.dot(u_iat_ref[...], wf[104:136, :],
                     preferred_element_type=jnp.float32)
        base = td[0:1, :] + b_fusion_ref[...] + ul[1:2, :] + ui[1:2, :]
        aux[0:1, :] = base
        aux[1:2, :] = td[1:2, :] - td[0:1, :]
        aux[2:3, :] = ul[0:1, :]
        aux[3:4, :] = ui[0:1, :]

    xb = x_ref[...]
    p = jnp.clip(xb[:, 0:1].astype(jnp.int32), 0, 255)
    f = jnp.clip(xb[:, 4:5].astype(jnp.int32), 0, 63)
    d = jnp.clip(xb[:, 3:4].astype(jnp.int32), 0, 1).astype(jnp.float32)

    oh_p = (p == jax.lax.broadcasted_iota(jnp.int32, (BT, 256), 1)
            ).astype(jnp.bfloat16)
    oh_f = (f == jax.lax.broadcasted_iota(jnp.int32, (BT, 64), 1)
            ).astype(jnp.bfloat16)

    h = jnp.dot(oh_p, tp_hi[...], preferred_element_type=jnp.float32)
    h = h + jnp.dot(oh_p, tp_lo[...], preferred_element_type=jnp.float32)
    h = h + jnp.dot(oh_f, tf_hi[...], preferred_element_type=jnp.float32)
    h = h + jnp.dot(oh_f, tf_lo[...], preferred_element_type=jnp.float32)
    h = (h + aux[0:1, :] + d * aux[1:2, :]
         + xb[:, 1:2] * aux[2:3, :] + xb[:, 2:3] * aux[3:4, :])

    mean = jnp.mean(h, axis=-1, keepdims=True)
    c = h - mean
    var = jnp.mean(c * c, axis=-1, keepdims=True)
    o_ref[...] = (c * jax.lax.rsqrt(var + 1e-5)) * gamma_ref[...] + beta_ref[...]


def kernel(x, emb_proto, emb_flags, emb_dir, w_len, b_len, w_iat, b_iat,
           w_fusion, b_fusion, gamma, beta):
    B, S, _ = x.shape
    n = B * S
    d_model = w_fusion.shape[1]
    xf = x.reshape(n, 5)

    emb_dir_p = jnp.pad(emb_dir, ((0, 8 - emb_dir.shape[0]), (0, 0)))
    u_len = jnp.concatenate(
        [w_len, b_len[None, :], jnp.zeros((6, 32), jnp.float32)], axis=0)
    u_iat = jnp.concatenate(
        [w_iat, b_iat[None, :], jnp.zeros((6, 32), jnp.float32)], axis=0)

    steps = n // BT
    grid = (steps,)
    full = lambda shape: pl.BlockSpec(shape, lambda i: (0, 0))
    out = pl.pallas_call(
        _body,
        grid=grid,
        in_specs=[
            pl.BlockSpec((BT, 5), lambda i: (i, 0)),
            full(emb_proto.shape),
            full(emb_flags.shape),
            full(emb_dir_p.shape),
            full(u_len.shape),
            full(u_iat.shape),
            full(w_fusion.shape),
            full((1, d_model)),
            full((1, d_model)),
            full((1, d_model)),
        ],
        out_specs=pl.BlockSpec((BT, d_model), lambda i: (i, 0)),
        out_shape=jax.ShapeDtypeStruct((n, d_model), jnp.float32),
        scratch_shapes=[
            pltpu.VMEM((256, 256), jnp.bfloat16),
            pltpu.VMEM((256, 256), jnp.bfloat16),
            pltpu.VMEM((64, 256), jnp.bfloat16),
            pltpu.VMEM((64, 256), jnp.bfloat16),
            pltpu.VMEM((8, 256), jnp.float32),
        ],
    )(xf, emb_proto, emb_flags, emb_dir_p, u_len, u_iat, w_fusion,
      b_fusion[None, :], gamma[None, :], beta[None, :])
    return out.reshape(B, S, d_model)
